# Initial kernel scaffold; baseline (speedup 1.0000x reference)
import functools

import jax
import jax.numpy as jnp
from jax import lax
from jax.experimental import pallas as pl
from jax.experimental.pallas import tpu as pltpu
from jax.experimental.pallas import tpu_sc as plsc

N = 6_400_000
S = 100_000
SP = 100_096
B = 2048
NBLK = N // B
NW = 32
NIT = -(-NBLK // NW)
CHUNK = 128
NCH = B // CHUNK
STRIPE = SP // 16
ZWORDS = STRIPE * 3


def _sc_body(qa_hbm, ra_hbm, seg_hbm, out_hbm,
             qa_v, ra_v, p_v, seg_v, zb_v, acc, sem):
    c = lax.axis_index("c")
    s = lax.axis_index("s")
    wid = c * 16 + s

    zero16 = jnp.zeros((16,), jnp.float32)

    def _zb(k, _):
        zb_v[pl.ds(k * 16, 16)] = zero16
        return 0

    lax.fori_loop(0, ZWORDS // 16, _zb, 0)
    pltpu.sync_copy(zb_v.reshape(STRIPE, 3), acc.at[pl.ds(s * STRIPE, STRIPE), :])
    plsc.subcore_barrier()

    iota16 = lax.iota(jnp.int32, 16)
    perms = [(iota16 + 16 * t) // 3 for t in range(3)]

    def _drain():
        for ch in range(NCH):
            pltpu.make_async_copy(
                p_v.at[pl.ds(ch * CHUNK * 3, CHUNK * 3)].reshape(CHUNK, 3),
                acc.at[seg_v.at[ch]],
                sem,
            ).wait()

    def _block(i, _):
        blk = wid + NW * i

        @pl.when(blk < NBLK)
        def _():
            @pl.when(i > 0)
            def _():
                _drain()

            base = blk * B
            pltpu.sync_copy(qa_hbm.at[pl.ds(base, B)], qa_v)
            pltpu.sync_copy(ra_hbm.at[pl.ds(base * 3, B * 3)], ra_v)
            pltpu.sync_copy(seg_hbm.at[pl.ds(blk * NCH, NCH)], seg_v)

            def _mul(j, _):
                q16 = qa_v[pl.ds(j * 16, 16)]
                for t in range(3):
                    off = j * 48 + t * 16
                    r = ra_v[pl.ds(off, 16)]
                    qp = jnp.take(q16, perms[t], mode="promise_in_bounds")
                    p_v[pl.ds(off, 16)] = qp * r
                return 0

            lax.fori_loop(0, B // 16, _mul, 0)

            for ch in range(NCH):
                pltpu.async_copy(
                    p_v.at[pl.ds(ch * CHUNK * 3, CHUNK * 3)].reshape(CHUNK, 3),
                    acc.at[seg_v.at[ch]],
                    sem,
                    add=True,
                )

        return 0

    lax.fori_loop(0, NIT, _block, 0)
    _drain()
    plsc.subcore_barrier()
    pltpu.sync_copy(acc.at[pl.ds(s * STRIPE, STRIPE), :],
                    out_hbm.at[c, pl.ds(s * STRIPE, STRIPE), :])


_sc_call = functools.partial(
    pl.kernel,
    out_type=jax.ShapeDtypeStruct((2, SP, 3), jnp.float32),
    mesh=plsc.VectorSubcoreMesh(core_axis_name="c", subcore_axis_name="s"),
    scratch_types=[
        pltpu.VMEM((B,), jnp.float32),
        pltpu.VMEM((B * 3,), jnp.float32),
        pltpu.VMEM((B * 3,), jnp.float32),
        pltpu.VMEM((NCH, CHUNK), jnp.int32),
        pltpu.VMEM((ZWORDS,), jnp.float32),
        pltpu.VMEM_SHARED((SP, 3), jnp.float32),
        pltpu.SemaphoreType.DMA,
    ],
)(_sc_body)


def _combine_body(a_ref, o_ref):
    o_ref[...] = a_ref[0] + a_ref[1]


_combine = pl.pallas_call(
    _combine_body,
    out_shape=jax.ShapeDtypeStruct((SP * 3 // 128, 128), jnp.float32),
)


def kernel(Qa, Ra, batch_seg):
    seg2d = batch_seg.astype(jnp.int32).reshape(N // CHUNK, CHUNK)
    ra_flat = Ra.reshape(-1)
    partial = _sc_call(Qa, ra_flat, seg2d)
    out = _combine(partial.reshape(2, SP * 3 // 128, 128))
    return out.reshape(SP * 3)[: S * 3].reshape(S, 3)


# speedup vs baseline: 2.0976x; 2.0976x over previous
import functools

import jax
import jax.numpy as jnp
from jax import lax
from jax.experimental import pallas as pl
from jax.experimental.pallas import tpu as pltpu
from jax.experimental.pallas import tpu_sc as plsc

N = 6_400_000
S = 100_000
SP = 106_496
B = 2048
NBLK = N // B
NW = 32
NIT = -(-NBLK // NW)
CHUNK = 128
NCH = B // CHUNK
STRIPE = SP // 16


def _sc_body(qa_hbm, ra_hbm, seg_hbm, out_hbm,
             qa_v, ra_v, px_v, py_v, pz_v, seg_v, zb_v,
             acc_x, acc_y, acc_z, sem):
    c = lax.axis_index("c")
    s = lax.axis_index("s")
    wid = c * 16 + s
    accs = (acc_x, acc_y, acc_z)
    ps = (px_v, py_v, pz_v)

    zero16 = jnp.zeros((16,), jnp.float32)

    def _zb(k, _):
        zb_v[pl.ds(k * 16, 16)] = zero16
        return 0

    lax.fori_loop(0, STRIPE // 16, _zb, 0)
    for a in accs:
        pltpu.sync_copy(zb_v, a.at[pl.ds(s * STRIPE, STRIPE)])
    plsc.subcore_barrier()

    iota3 = lax.iota(jnp.int32, 16) * 3

    def _drain():
        for ch in range(NCH):
            for t, a in enumerate(accs):
                pltpu.make_async_copy(
                    ps[t].at[pl.ds(ch * CHUNK, CHUNK)],
                    a.at[seg_v.at[pl.ds(ch * CHUNK, CHUNK)]],
                    sem,
                ).wait()

    def _block(i, _):
        blk = wid + NW * i

        @pl.when(blk < NBLK)
        def _():
            @pl.when(i > 0)
            def _():
                _drain()

            base = blk * B
            pltpu.sync_copy(qa_hbm.at[pl.ds(base, B)], qa_v)
            pltpu.sync_copy(ra_hbm.at[pl.ds(base * 3, B * 3)], ra_v)
            pltpu.sync_copy(seg_hbm.at[pl.ds(base, B)], seg_v)

            def _mul(j, _):
                jbase = j * 16
                q16 = qa_v[pl.ds(jbase, 16)]
                ridx = jbase * 3 + iota3
                for t in range(3):
                    r = plsc.load_gather(ra_v, [ridx + t])
                    ps[t][pl.ds(jbase, 16)] = q16 * r
                return 0

            lax.fori_loop(0, B // 16, _mul, 0)

            for ch in range(NCH):
                for t, a in enumerate(accs):
                    pltpu.async_copy(
                        ps[t].at[pl.ds(ch * CHUNK, CHUNK)],
                        a.at[seg_v.at[pl.ds(ch * CHUNK, CHUNK)]],
                        sem,
                        add=True,
                    )

        return 0

    lax.fori_loop(0, NIT, _block, 0)
    _drain()
    plsc.subcore_barrier()
    for t, a in enumerate(accs):
        pltpu.sync_copy(a.at[pl.ds(s * STRIPE, STRIPE)],
                        out_hbm.at[pl.ds((c * 3 + t) * SP + s * STRIPE, STRIPE)])


_sc_call = functools.partial(
    pl.kernel,
    out_type=jax.ShapeDtypeStruct((6 * SP,), jnp.float32),
    mesh=plsc.VectorSubcoreMesh(core_axis_name="c", subcore_axis_name="s"),
    compiler_params=pltpu.CompilerParams(needs_layout_passes=False),
    scratch_types=[
        pltpu.VMEM((B,), jnp.float32),
        pltpu.VMEM((B * 3,), jnp.float32),
        pltpu.VMEM((B,), jnp.float32),
        pltpu.VMEM((B,), jnp.float32),
        pltpu.VMEM((B,), jnp.float32),
        pltpu.VMEM((B,), jnp.int32),
        pltpu.VMEM((STRIPE,), jnp.float32),
        pltpu.VMEM_SHARED((SP,), jnp.float32),
        pltpu.VMEM_SHARED((SP,), jnp.float32),
        pltpu.VMEM_SHARED((SP,), jnp.float32),
        pltpu.SemaphoreType.DMA,
    ],
)(_sc_body)


def _combine_body(a_ref, o_ref):
    o_ref[...] = a_ref[0] + a_ref[1]


_combine = pl.pallas_call(
    _combine_body,
    out_shape=jax.ShapeDtypeStruct((3, SP), jnp.float32),
)


def kernel(Qa, Ra, batch_seg):
    seg32 = batch_seg.astype(jnp.int32)
    ra_flat = Ra.reshape(-1)
    partial = _sc_call(Qa, ra_flat, seg32)
    out = _combine(partial.reshape(2, 3, SP))
    return out[:, :S].T


# speedup vs baseline: 24.3646x; 11.6155x over previous
import functools

import jax
import jax.numpy as jnp
from jax import lax
from jax.experimental import pallas as pl
from jax.experimental.pallas import tpu as pltpu
from jax.experimental.pallas import tpu_sc as plsc

N = 6_400_000
S = 100_000
SP = 106_496
B = 2048
NBLK = N // B
NW = 32
NIT = -(-NBLK // NW)
STRIPE = SP // 16


def _sc_body(qa_hbm, rx_hbm, ry_hbm, rz_hbm, seg_hbm, out_hbm,
             qa_v, rx_v, ry_v, rz_v, px_v, py_v, pz_v, seg_v, zb_v,
             acc_x, acc_y, acc_z, sem, sem_in):
    c = lax.axis_index("c")
    s = lax.axis_index("s")
    wid = c * 16 + s
    accs = (acc_x, acc_y, acc_z)
    rs = (rx_v, ry_v, rz_v)
    ps = (px_v, py_v, pz_v)
    r_hbms = (rx_hbm, ry_hbm, rz_hbm)

    zero16 = jnp.zeros((16,), jnp.float32)

    def _zb(k, _):
        zb_v[pl.ds(k * 16, 16)] = zero16
        return 0

    lax.fori_loop(0, STRIPE // 16, _zb, 0)
    for a in accs:
        pltpu.sync_copy(zb_v, a.at[pl.ds(s * STRIPE, STRIPE)])
    plsc.subcore_barrier()

    def _drain():
        for t, a in enumerate(accs):
            pltpu.make_async_copy(ps[t], a.at[seg_v], sem).wait()

    def _block(i, _):
        blk = wid + NW * i

        @pl.when(blk < NBLK)
        def _():
            @pl.when(i > 0)
            def _():
                _drain()

            base = blk * B
            copies = [pltpu.async_copy(qa_hbm.at[pl.ds(base, B)], qa_v, sem_in),
                      pltpu.async_copy(seg_hbm.at[pl.ds(base, B)], seg_v, sem_in)]
            for t in range(3):
                copies.append(
                    pltpu.async_copy(r_hbms[t].at[pl.ds(base, B)], rs[t], sem_in))
            for cp in copies:
                cp.wait()

            def _mul(j, _):
                jbase = j * 16
                q16 = qa_v[pl.ds(jbase, 16)]
                for t in range(3):
                    ps[t][pl.ds(jbase, 16)] = q16 * rs[t][pl.ds(jbase, 16)]
                return 0

            lax.fori_loop(0, B // 16, _mul, 0)

            for t, a in enumerate(accs):
                pltpu.async_copy(ps[t], a.at[seg_v], sem, add=True)

        return 0

    lax.fori_loop(0, NIT, _block, 0)
    _drain()
    plsc.subcore_barrier()
    for t, a in enumerate(accs):
        pltpu.sync_copy(a.at[pl.ds(s * STRIPE, STRIPE)],
                        out_hbm.at[pl.ds((c * 3 + t) * SP + s * STRIPE, STRIPE)])


_sc_call = functools.partial(
    pl.kernel,
    out_type=jax.ShapeDtypeStruct((6 * SP,), jnp.float32),
    mesh=plsc.VectorSubcoreMesh(core_axis_name="c", subcore_axis_name="s"),
    compiler_params=pltpu.CompilerParams(needs_layout_passes=False),
    scratch_types=[
        pltpu.VMEM((B,), jnp.float32),
        pltpu.VMEM((B,), jnp.float32),
        pltpu.VMEM((B,), jnp.float32),
        pltpu.VMEM((B,), jnp.float32),
        pltpu.VMEM((B,), jnp.float32),
        pltpu.VMEM((B,), jnp.float32),
        pltpu.VMEM((B,), jnp.float32),
        pltpu.VMEM((B,), jnp.int32),
        pltpu.VMEM((STRIPE,), jnp.float32),
        pltpu.VMEM_SHARED((SP,), jnp.float32),
        pltpu.VMEM_SHARED((SP,), jnp.float32),
        pltpu.VMEM_SHARED((SP,), jnp.float32),
        pltpu.SemaphoreType.DMA,
        pltpu.SemaphoreType.DMA,
    ],
)(_sc_body)


def _combine_body(a_ref, o_ref):
    o_ref[...] = a_ref[0] + a_ref[1]


_combine = pl.pallas_call(
    _combine_body,
    out_shape=jax.ShapeDtypeStruct((3, SP), jnp.float32),
)


def kernel(Qa, Ra, batch_seg):
    seg32 = batch_seg.astype(jnp.int32)
    rx, ry, rz = Ra[:, 0], Ra[:, 1], Ra[:, 2]
    partial = _sc_call(Qa, rx, ry, rz, seg32)
    out = _combine(partial.reshape(2, 3, SP))
    return out[:, :S].T


# speedup vs baseline: 27.0962x; 1.1121x over previous
import functools

import jax
import jax.numpy as jnp
from jax import lax
from jax.experimental import pallas as pl
from jax.experimental.pallas import tpu as pltpu
from jax.experimental.pallas import tpu_sc as plsc

N = 6_400_000
S = 100_000
SP = 106_496
B = 4000
NBLK = N // B
NW = 32
NIT = NBLK // NW
STRIPE = SP // 16
CB = B + 160


def _sc_body(qa_hbm, rx_hbm, ry_hbm, rz_hbm, seg_hbm, out_hbm,
             qa_v, rx_v, ry_v, rz_v, seg_v, cs_v,
             cseg_v, cpx_v, cpy_v, cpz_v, zb_v,
             acc_x, acc_y, acc_z, sem, sem_in):
    c = lax.axis_index("c")
    s = lax.axis_index("s")
    wid = c * 16 + s
    accs = (acc_x, acc_y, acc_z)
    rs = (rx_v, ry_v, rz_v)
    cps = (cpx_v, cpy_v, cpz_v)
    r_hbms = (rx_hbm, ry_hbm, rz_hbm)

    iota = lax.iota(jnp.int32, 16)
    zero16 = jnp.zeros((16,), jnp.float32)
    neg1_16 = jnp.full((16,), -1, jnp.int32)

    def _zb(k, _):
        zb_v[pl.ds(k * 16, 16)] = zero16
        return 0

    lax.fori_loop(0, STRIPE // 16, _zb, 0)
    for a in accs:
        pltpu.sync_copy(zb_v, a.at[pl.ds(s * STRIPE, STRIPE)])
    plsc.subcore_barrier()

    seg_v[pl.ds(0, 16)] = neg1_16
    seg_v[pl.ds(B + 16, 16)] = neg1_16

    dump16 = S + wid * 16 + iota

    def _block(i, prev_trips):
        blk = wid + NW * i
        base = blk * B

        def _drain(k, _):
            for t, a in enumerate(accs):
                pltpu.make_async_copy(
                    cps[t].at[pl.ds(k * 128, 128)],
                    a.at[cseg_v.at[pl.ds(k * 128, 128)]],
                    sem,
                ).wait()
            return 0

        lax.fori_loop(0, prev_trips, _drain, 0)

        copies = [pltpu.async_copy(qa_hbm.at[pl.ds(base, B)], qa_v, sem_in),
                  pltpu.async_copy(seg_hbm.at[pl.ds(base, B)],
                                   seg_v.at[pl.ds(16, B)], sem_in)]
        for t in range(3):
            copies.append(
                pltpu.async_copy(r_hbms[t].at[pl.ds(base, B)], rs[t], sem_in))
        for cp in copies:
            cp.wait()

        def _mul(j, cursor):
            jb = j * 16
            q16 = qa_v[pl.ds(jb, 16)]
            seg = seg_v[pl.ds(jb + 16, 16)]
            segn = plsc.load_gather(seg_v, [iota + (jb + 17)])
            segp = plsc.load_gather(seg_v, [iota + (jb + 15)])
            end = jnp.logical_or(seg != segn, iota == 15)
            sp = seg != segp
            rank = plsc.cumsum(jnp.where(sp, 1, 0))
            pe = plsc.cummax(jnp.where(sp, iota - 1, -1))
            pe_ok = pe >= 0
            pec = jnp.maximum(pe, 0)
            rsum = []
            for t in range(3):
                cs = plsc.cumsum(q16 * rs[t][pl.ds(jb, 16)])
                cs_v[pl.ds(t * 16, 16)] = cs
                g = plsc.load_gather(cs_v, [pec + t * 16])
                rsum.append(cs - jnp.where(pe_ok, g, 0.0))
            idxv = cursor + rank
            m_new = jnp.logical_and(end, rank > 0)
            m_bnd = jnp.logical_and(end, rank == 0)
            plsc.store_scatter(cseg_v, [idxv], seg, mask=m_new)
            for t in range(3):
                plsc.store_scatter(cps[t], [idxv], rsum[t], mask=m_new)
                plsc.addupdate_scatter(cps[t], [idxv], rsum[t], mask=m_bnd)
            nst = plsc.all_reduce_population_count(sp)
            return cursor + nst

        cursor = lax.fori_loop(0, B // 16, _mul, neg1_16)
        count = jnp.max(cursor) + 1
        fl = (count // 16) * 16
        for k in range(10):
            pos = fl + 16 * k + iota
            plsc.store_scatter(cseg_v, [pos], dump16, mask=pos >= count)
        trips = (count + 127) // 128

        def _scat(k, _):
            for t, a in enumerate(accs):
                pltpu.async_copy(
                    cps[t].at[pl.ds(k * 128, 128)],
                    a.at[cseg_v.at[pl.ds(k * 128, 128)]],
                    sem,
                    add=True,
                )
            return 0

        lax.fori_loop(0, trips, _scat, 0)
        return trips

    final_trips = lax.fori_loop(0, NIT, _block, jnp.int32(0))

    def _drain_last(k, _):
        for t, a in enumerate(accs):
            pltpu.make_async_copy(
                cps[t].at[pl.ds(k * 128, 128)],
                a.at[cseg_v.at[pl.ds(k * 128, 128)]],
                sem,
            ).wait()
        return 0

    lax.fori_loop(0, final_trips, _drain_last, 0)
    plsc.subcore_barrier()
    for t, a in enumerate(accs):
        pltpu.sync_copy(a.at[pl.ds(s * STRIPE, STRIPE)],
                        out_hbm.at[pl.ds((c * 3 + t) * SP + s * STRIPE, STRIPE)])


_sc_call = functools.partial(
    pl.kernel,
    out_type=jax.ShapeDtypeStruct((6 * SP,), jnp.float32),
    mesh=plsc.VectorSubcoreMesh(core_axis_name="c", subcore_axis_name="s"),
    compiler_params=pltpu.CompilerParams(needs_layout_passes=False),
    scratch_types=[
        pltpu.VMEM((B,), jnp.float32),
        pltpu.VMEM((B,), jnp.float32),
        pltpu.VMEM((B,), jnp.float32),
        pltpu.VMEM((B,), jnp.float32),
        pltpu.VMEM((B + 32,), jnp.int32),
        pltpu.VMEM((48,), jnp.float32),
        pltpu.VMEM((CB,), jnp.int32),
        pltpu.VMEM((CB,), jnp.float32),
        pltpu.VMEM((CB,), jnp.float32),
        pltpu.VMEM((CB,), jnp.float32),
        pltpu.VMEM((STRIPE,), jnp.float32),
        pltpu.VMEM_SHARED((SP,), jnp.float32),
        pltpu.VMEM_SHARED((SP,), jnp.float32),
        pltpu.VMEM_SHARED((SP,), jnp.float32),
        pltpu.SemaphoreType.DMA,
        pltpu.SemaphoreType.DMA,
    ],
)(_sc_body)


def _combine_body(a_ref, o_ref):
    o_ref[...] = a_ref[0] + a_ref[1]


_combine = pl.pallas_call(
    _combine_body,
    out_shape=jax.ShapeDtypeStruct((3, SP), jnp.float32),
)


def kernel(Qa, Ra, batch_seg):
    seg32 = batch_seg.astype(jnp.int32)
    rx, ry, rz = Ra[:, 0], Ra[:, 1], Ra[:, 2]
    partial = _sc_call(Qa, rx, ry, rz, seg32)
    out = _combine(partial.reshape(2, 3, SP))
    return out[:, :S].T
